# Initial kernel scaffold; baseline (speedup 1.0000x reference)
import jax
import jax.numpy as jnp
from jax.experimental import pallas as pl


def kernel(rna, protein, params, edge_index):
    raise NotImplementedError("write your pallas kernel here")


# speedup vs baseline: 3.3047x; 3.3047x over previous
import functools

import jax
import jax.numpy as jnp
from jax import lax
from jax.experimental import pallas as pl
from jax.experimental.pallas import tpu as pltpu
from jax.experimental.pallas import tpu_sc as plsc

N = 50000
E = 800000
D = 128
H = 64
O = 64

NC = 2
NS = 16
NPAD = 50176
NP_TILE = NPAD // NS
EPAD = 819200
EROWS = EPAD // 128
SUB = 128
K = 8
ROWS_PER_TILE = EROWS // NS
MACROS = ROWS_PER_TILE // K

R = 2000
GRID = N // R

f32 = jnp.float32
i32 = jnp.int32


def _relu(v):
    return jnp.maximum(v, 0.0)


def _b(ref):
    return ref[...][0:1, :]


def _tc1_body(rna, prot, wr1, br1, wr2, br2, wp1, bp1, wp2, bp2,
              iw1a, iw1b, ib1, iw2, ib2,
              piw, pib, mtop, pjw, pjb, mbot, mb1,
              x_out, a_out, b_out):
    er = _relu(rna[...] @ wr1[...] + _b(br1)) @ wr2[...] + _b(br2)
    ep = _relu(prot[...] @ wp1[...] + _b(bp1)) @ wp2[...] + _b(bp2)
    h = _relu(er @ iw1a[...] + ep @ iw1b[...] + _b(ib1))
    x = h @ iw2[...] + _b(ib2)
    x_out[...] = x
    a_out[...] = _relu(x @ piw[...] + _b(pib)) @ mtop[...] + _b(mb1)
    b_out[...] = _relu(x @ pjw[...] + _b(pjb)) @ mbot[...]


def _mid_body_next(s0, s1, s2, s3, c0, c1, x,
                   w20, w21, w22, w23, b2, u1a, u1b, ub1, uw2, ub2,
                   piw, pib, mtop, pjw, pjb, mbot, mb1,
                   x_out, a_out, b_out):
    counts = c0[...][:, 0:1] + c1[...][:, 0:1]
    aggr = (s0[...] @ w20[...] + s1[...] @ w21[...]
            + s2[...] @ w22[...] + s3[...] @ w23[...] + counts * _b(b2))
    uh = _relu(aggr @ u1a[...] + x[...] @ u1b[...] + _b(ub1))
    xn = _relu(uh @ uw2[...] + _b(ub2))
    x_out[...] = xn
    a_out[...] = _relu(xn @ piw[...] + _b(pib)) @ mtop[...] + _b(mb1)
    b_out[...] = _relu(xn @ pjw[...] + _b(pjb)) @ mbot[...]


def _mid_body_last(s0, s1, s2, s3, c0, c1, x,
                   w20, w21, w22, w23, b2, u1a, u1b, ub1, uw2, ub2,
                   x_out):
    counts = c0[...][:, 0:1] + c1[...][:, 0:1]
    aggr = (s0[...] @ w20[...] + s1[...] @ w21[...]
            + s2[...] @ w22[...] + s3[...] @ w23[...] + counts * _b(b2))
    uh = _relu(aggr @ u1a[...] + x[...] @ u1b[...] + _b(ub1))
    x_out[...] = _relu(uh @ uw2[...] + _b(ub2))


def _row_spec(cols):
    return pl.BlockSpec((R, cols), lambda i: (i, 0))


def _full_spec(shape):
    return pl.BlockSpec(shape, lambda i: tuple(0 for _ in shape))


def _tile_bias(b):
    return jnp.tile(b.reshape(1, -1), (8, 1))


def _call_tc(body, row_in_cols, weight_shapes, n_out):
    in_specs = ([_row_spec(c) for c in row_in_cols]
                + [_full_spec(s) for s in weight_shapes])
    out_specs = [_row_spec(64) for _ in range(n_out)]
    out_shape = [jax.ShapeDtypeStruct((N, 64), f32) for _ in range(n_out)]
    return pl.pallas_call(
        body,
        grid=(GRID,),
        in_specs=in_specs,
        out_specs=out_specs if n_out > 1 else out_specs[0],
        out_shape=out_shape if n_out > 1 else out_shape[0],
    )


@functools.lru_cache(maxsize=None)
def _build_edge_kernel():
    mesh = plsc.VectorSubcoreMesh(core_axis_name="c", subcore_axis_name="s",
                                  num_cores=NC, num_subcores=NS)
    return pl.kernel(
        _edge_body,
        out_type=jax.ShapeDtypeStruct((4 * NPAD, 16), f32),
        mesh=mesh,
        compiler_params=pltpu.CompilerParams(use_tc_tiling_on_sc=False),
        scratch_types=[
            pltpu.VMEM((K, SUB), i32),
            pltpu.VMEM((K, SUB), i32),
            pltpu.VMEM((K, SUB), i32),
            pltpu.VMEM((K * SUB, 16), f32),
            pltpu.VMEM((K * SUB, 16), f32),
            pltpu.VMEM_SHARED((NPAD, 16), f32),
            pltpu.SemaphoreType.DMA,
            pltpu.SemaphoreType.DMA,
        ],
    )


def _edge_body(a_tab, b_tab, dst2d, src2d, zeros16, out,
               idx_d, idx_ga, idx_gs, abuf, bbuf, acc, sem_a, sem_b):
    c = lax.axis_index("c")
    s = lax.axis_index("s")
    row0 = s * ROWS_PER_TILE

    for p in range(2):
        qoff = (c * 2 + p) * NPAD

        pltpu.sync_copy(zeros16, acc.at[pl.ds(s * NP_TILE, NP_TILE)])
        plsc.subcore_barrier()

        def macro(m, carry, qoff=qoff):
            base = row0 + m * K
            pltpu.sync_copy(dst2d.at[pl.ds(base, K)], idx_d)
            pltpu.sync_copy(src2d.at[pl.ds(base, K)], idx_gs)

            def adj(j, carry2):
                for k in range(SUB // 16):
                    sl = pl.ds(k * 16, 16)
                    idx_ga[j, sl] = idx_d[j, sl] + qoff
                    idx_gs[j, sl] = idx_gs[j, sl] + qoff
                return carry2
            lax.fori_loop(0, K, adj, 0)

            copies = []
            for j in range(K):
                copies.append(pltpu.async_copy(
                    a_tab.at[idx_ga.at[j]], abuf.at[pl.ds(j * SUB, SUB)],
                    sem_a))
                copies.append(pltpu.async_copy(
                    b_tab.at[idx_gs.at[j]], bbuf.at[pl.ds(j * SUB, SUB)],
                    sem_b))
            for cp in copies:
                cp.wait()

            def comp(r, carry2):
                sl = pl.ds(0, 16)
                abuf[r, sl] = jnp.maximum(abuf[r, sl] + bbuf[r, sl], 0.0)
                return carry2
            lax.fori_loop(0, K * SUB, comp, 0)

            for j in range(K):
                pltpu.sync_copy(abuf.at[pl.ds(j * SUB, SUB)],
                                acc.at[idx_d.at[j]], add=True)
            return carry

        lax.fori_loop(0, MACROS, macro, 0)
        plsc.subcore_barrier()
        pltpu.sync_copy(acc.at[pl.ds(s * NP_TILE, NP_TILE)],
                        out.at[pl.ds(qoff + s * NP_TILE, NP_TILE)])


@functools.lru_cache(maxsize=None)
def _build_count_kernel():
    mesh = plsc.VectorSubcoreMesh(core_axis_name="c", subcore_axis_name="s",
                                  num_cores=NC, num_subcores=NS)
    return pl.kernel(
        _count_body,
        out_type=jax.ShapeDtypeStruct((2 * NPAD, 16), f32),
        mesh=mesh,
        compiler_params=pltpu.CompilerParams(use_tc_tiling_on_sc=False),
        scratch_types=[
            pltpu.VMEM((K, SUB), i32),
            pltpu.VMEM((SUB, 16), f32),
            pltpu.VMEM_SHARED((NPAD, 16), f32),
        ],
    )


def _count_body(dst2d, zeros16, out, idx_d, ones, acc):
    c = lax.axis_index("c")
    s = lax.axis_index("s")

    pltpu.sync_copy(zeros16, acc.at[pl.ds(s * NP_TILE, NP_TILE)])

    def fill(r, carry):
        ones[r, pl.ds(0, 16)] = jnp.full((16,), 1.0, f32)
        return carry
    lax.fori_loop(0, SUB, fill, 0)
    plsc.subcore_barrier()

    w = c * NS + s
    row0 = w * (EROWS // (NC * NS))

    def macro(m, carry):
        pltpu.sync_copy(dst2d.at[pl.ds(row0 + m * K, K)], idx_d)
        for j in range(K):
            pltpu.sync_copy(ones, acc.at[idx_d.at[j]], add=True)
        return carry
    lax.fori_loop(0, (EROWS // (NC * NS)) // K, macro, 0)
    plsc.subcore_barrier()
    pltpu.sync_copy(acc.at[pl.ds(s * NP_TILE, NP_TILE)],
                    out.at[pl.ds(c * NPAD + s * NP_TILE, NP_TILE)])


def _pad_tab(t):
    qs = [jnp.pad(t[:, q * 16:(q + 1) * 16], ((0, NPAD - N), (0, 0)))
          for q in range(4)]
    return jnp.concatenate(qs, axis=0)


def kernel(rna, protein, params, edge_index):
    p = params

    src = edge_index[0]
    dst = edge_index[1]
    pad_ids = jnp.full((EPAD - E,), N, i32)
    dst2d = jnp.concatenate([dst, pad_ids]).reshape(EROWS, SUB)
    src2d = jnp.concatenate([src, pad_ids]).reshape(EROWS, SUB)
    zeros16 = jnp.zeros((NP_TILE, 16), f32)

    tc1 = _call_tc(
        _tc1_body, [D, D],
        [(D, H), (8, H), (H, H), (8, H),
         (D, H), (8, H), (H, H), (8, H),
         (H, H), (H, H), (8, H), (H, O), (8, O),
         (O, H), (8, H), (H, H), (O, H), (8, H), (H, H), (8, H)],
        3)
    x0, a0, b0 = tc1(
        rna, protein,
        p['enc_rna_w1'], _tile_bias(p['enc_rna_b1']),
        p['enc_rna_w2'], _tile_bias(p['enc_rna_b2']),
        p['enc_protein_w1'], _tile_bias(p['enc_protein_b1']),
        p['enc_protein_w2'], _tile_bias(p['enc_protein_b2']),
        p['int_w1'][:H], p['int_w1'][H:], _tile_bias(p['int_b1']),
        p['int_w2'], _tile_bias(p['int_b2']),
        p['g0_proj_i_w'][:O], _tile_bias(p['g0_proj_i_b']),
        p['g0_msg_w1'][:H],
        p['g0_proj_j_w'][:O], _tile_bias(p['g0_proj_j_b']),
        p['g0_msg_w1'][H:], _tile_bias(p['g0_msg_b1']))

    cnt = _build_count_kernel()(dst2d, zeros16)
    c0 = cnt[:N]
    c1 = cnt[NPAD:NPAD + N]

    s0 = _build_edge_kernel()(_pad_tab(a0), _pad_tab(b0), dst2d, src2d,
                              zeros16)

    def quarters(sarr):
        return [sarr[q * NPAD:q * NPAD + N] for q in range(4)]

    def w2_quarters(w2):
        return [w2[q * 16:(q + 1) * 16] for q in range(4)]

    tc2 = _call_tc(
        _mid_body_next, [16, 16, 16, 16, 16, 16, 64],
        [(16, H), (16, H), (16, H), (16, H), (8, H),
         (H, H), (O, H), (8, H), (H, O), (8, O),
         (O, H), (8, H), (H, H), (O, H), (8, H), (H, H), (8, H)],
        3)
    x1, a1, b1 = tc2(
        *quarters(s0), c0, c1, x0,
        *w2_quarters(p['g0_msg_w2']), _tile_bias(p['g0_msg_b2']),
        p['g0_upd_w1'][:H], p['g0_upd_w1'][H:], _tile_bias(p['g0_upd_b1']),
        p['g0_upd_w2'], _tile_bias(p['g0_upd_b2']),
        p['g1_proj_i_w'][:O], _tile_bias(p['g1_proj_i_b']),
        p['g1_msg_w1'][:H],
        p['g1_proj_j_w'][:O], _tile_bias(p['g1_proj_j_b']),
        p['g1_msg_w1'][H:], _tile_bias(p['g1_msg_b1']))

    s1 = _build_edge_kernel()(_pad_tab(a1), _pad_tab(b1), dst2d, src2d,
                              zeros16)

    tc3 = _call_tc(
        _mid_body_last, [16, 16, 16, 16, 16, 16, 64],
        [(16, H), (16, H), (16, H), (16, H), (8, H),
         (H, H), (O, H), (8, H), (H, O), (8, O)],
        1)
    x2 = tc3(
        *quarters(s1), c0, c1, x1,
        *w2_quarters(p['g1_msg_w2']), _tile_bias(p['g1_msg_b2']),
        p['g1_upd_w1'][:H], p['g1_upd_w1'][H:], _tile_bias(p['g1_upd_b1']),
        p['g1_upd_w2'], _tile_bias(p['g1_upd_b2']))
    return x2


# speedup vs baseline: 4.1824x; 1.2656x over previous
import functools

import jax
import jax.numpy as jnp
from jax import lax
from jax.experimental import pallas as pl
from jax.experimental.pallas import tpu as pltpu
from jax.experimental.pallas import tpu_sc as plsc

N = 50000
E = 800000
D = 128
H = 64
O = 64

NC = 2
NS = 16
NPAD = 50176
NP_TILE = NPAD // NS
EPAD = 819200
EROWS = EPAD // 128
SUB = 128
K = 8
ROWS_PER_TILE = EROWS // NS
MACROS = ROWS_PER_TILE // K

R = 2000
GRID = N // R

f32 = jnp.float32
i32 = jnp.int32


def _relu(v):
    return jnp.maximum(v, 0.0)


def _b(ref):
    return ref[...][0:1, :]


def _tc1_body(rna, prot, wr1, br1, wr2, br2, wp1, bp1, wp2, bp2,
              iw1a, iw1b, ib1, iw2, ib2,
              piw, pib, mtop, pjw, pjb, mbot, mb1,
              x_out, a_out, b_out):
    er = _relu(rna[...] @ wr1[...] + _b(br1)) @ wr2[...] + _b(br2)
    ep = _relu(prot[...] @ wp1[...] + _b(bp1)) @ wp2[...] + _b(bp2)
    h = _relu(er @ iw1a[...] + ep @ iw1b[...] + _b(ib1))
    x = h @ iw2[...] + _b(ib2)
    x_out[...] = x
    a_out[...] = _relu(x @ piw[...] + _b(pib)) @ mtop[...] + _b(mb1)
    b_out[...] = _relu(x @ pjw[...] + _b(pjb)) @ mbot[...]


def _mid_body_next(s0, s1, s2, s3, c0, c1, x,
                   w20, w21, w22, w23, b2, u1a, u1b, ub1, uw2, ub2,
                   piw, pib, mtop, pjw, pjb, mbot, mb1,
                   x_out, a_out, b_out):
    counts = c0[...][:, 0:1] + c1[...][:, 0:1]
    aggr = (s0[...] @ w20[...] + s1[...] @ w21[...]
            + s2[...] @ w22[...] + s3[...] @ w23[...] + counts * _b(b2))
    uh = _relu(aggr @ u1a[...] + x[...] @ u1b[...] + _b(ub1))
    xn = _relu(uh @ uw2[...] + _b(ub2))
    x_out[...] = xn
    a_out[...] = _relu(xn @ piw[...] + _b(pib)) @ mtop[...] + _b(mb1)
    b_out[...] = _relu(xn @ pjw[...] + _b(pjb)) @ mbot[...]


def _mid_body_last(s0, s1, s2, s3, c0, c1, x,
                   w20, w21, w22, w23, b2, u1a, u1b, ub1, uw2, ub2,
                   x_out):
    counts = c0[...][:, 0:1] + c1[...][:, 0:1]
    aggr = (s0[...] @ w20[...] + s1[...] @ w21[...]
            + s2[...] @ w22[...] + s3[...] @ w23[...] + counts * _b(b2))
    uh = _relu(aggr @ u1a[...] + x[...] @ u1b[...] + _b(ub1))
    x_out[...] = _relu(uh @ uw2[...] + _b(ub2))


def _row_spec(cols):
    return pl.BlockSpec((R, cols), lambda i: (i, 0))


def _full_spec(shape):
    return pl.BlockSpec(shape, lambda i: tuple(0 for _ in shape))


def _tile_bias(b):
    return jnp.tile(b.reshape(1, -1), (8, 1))


def _call_tc(body, row_in_cols, weight_shapes, n_out):
    in_specs = ([_row_spec(c) for c in row_in_cols]
                + [_full_spec(s) for s in weight_shapes])
    out_specs = [_row_spec(64) for _ in range(n_out)]
    out_shape = [jax.ShapeDtypeStruct((N, 64), f32) for _ in range(n_out)]
    return pl.pallas_call(
        body,
        grid=(GRID,),
        in_specs=in_specs,
        out_specs=out_specs if n_out > 1 else out_specs[0],
        out_shape=out_shape if n_out > 1 else out_shape[0],
    )


@functools.lru_cache(maxsize=None)
def _build_edge_kernel():
    mesh = plsc.VectorSubcoreMesh(core_axis_name="c", subcore_axis_name="s",
                                  num_cores=NC, num_subcores=NS)
    return pl.kernel(
        _edge_body,
        out_type=jax.ShapeDtypeStruct((4 * NPAD, 16), f32),
        mesh=mesh,
        compiler_params=pltpu.CompilerParams(use_tc_tiling_on_sc=False),
        scratch_types=[
            pltpu.VMEM((K, SUB), i32),
            pltpu.VMEM((K, SUB), i32),
            pltpu.VMEM((K, SUB), i32),
            pltpu.VMEM((K * SUB, 16), f32),
            pltpu.VMEM((K * SUB, 16), f32),
            pltpu.VMEM_SHARED((NPAD, 16), f32),
            pltpu.SemaphoreType.DMA,
            pltpu.SemaphoreType.DMA,
        ],
    )


def _edge_body(a_tab, b_tab, dst2d, src2d, zeros16, out,
               idx_d, idx_ga, idx_gs, abuf, bbuf, acc, sem_a, sem_b):
    c = lax.axis_index("c")
    s = lax.axis_index("s")
    row0 = s * ROWS_PER_TILE

    for p in range(2):
        qoff = (c * 2 + p) * NPAD

        pltpu.sync_copy(zeros16, acc.at[pl.ds(s * NP_TILE, NP_TILE)])
        plsc.subcore_barrier()

        def macro(m, carry, qoff=qoff):
            base = row0 + m * K
            pltpu.sync_copy(dst2d.at[pl.ds(base, K)], idx_d)
            pltpu.sync_copy(src2d.at[pl.ds(base, K)], idx_gs)

            @plsc.parallel_loop(0, K)
            def adj(j):
                for k in range(SUB // 16):
                    sl = pl.ds(k * 16, 16)
                    idx_ga[j, sl] = idx_d[j, sl] + qoff
                    idx_gs[j, sl] = idx_gs[j, sl] + qoff

            copies = []
            for j in range(K):
                copies.append(pltpu.async_copy(
                    a_tab.at[idx_ga.at[j]], abuf.at[pl.ds(j * SUB, SUB)],
                    sem_a))
                copies.append(pltpu.async_copy(
                    b_tab.at[idx_gs.at[j]], bbuf.at[pl.ds(j * SUB, SUB)],
                    sem_b))
            for cp in copies:
                cp.wait()

            @plsc.parallel_loop(0, K * SUB, unroll=8)
            def comp(r):
                sl = pl.ds(0, 16)
                abuf[r, sl] = jnp.maximum(abuf[r, sl] + bbuf[r, sl], 0.0)

            for j in range(K):
                pltpu.sync_copy(abuf.at[pl.ds(j * SUB, SUB)],
                                acc.at[idx_d.at[j]], add=True)
            return carry

        lax.fori_loop(0, MACROS, macro, 0)
        plsc.subcore_barrier()
        pltpu.sync_copy(acc.at[pl.ds(s * NP_TILE, NP_TILE)],
                        out.at[pl.ds(qoff + s * NP_TILE, NP_TILE)])


@functools.lru_cache(maxsize=None)
def _build_count_kernel():
    mesh = plsc.VectorSubcoreMesh(core_axis_name="c", subcore_axis_name="s",
                                  num_cores=NC, num_subcores=NS)
    return pl.kernel(
        _count_body,
        out_type=jax.ShapeDtypeStruct((2 * NPAD, 16), f32),
        mesh=mesh,
        compiler_params=pltpu.CompilerParams(use_tc_tiling_on_sc=False),
        scratch_types=[
            pltpu.VMEM((K, SUB), i32),
            pltpu.VMEM((SUB, 16), f32),
            pltpu.VMEM_SHARED((NPAD, 16), f32),
        ],
    )


def _count_body(dst2d, zeros16, out, idx_d, ones, acc):
    c = lax.axis_index("c")
    s = lax.axis_index("s")

    pltpu.sync_copy(zeros16, acc.at[pl.ds(s * NP_TILE, NP_TILE)])

    def fill(r, carry):
        ones[r, pl.ds(0, 16)] = jnp.full((16,), 1.0, f32)
        return carry
    lax.fori_loop(0, SUB, fill, 0)
    plsc.subcore_barrier()

    w = c * NS + s
    row0 = w * (EROWS // (NC * NS))

    def macro(m, carry):
        pltpu.sync_copy(dst2d.at[pl.ds(row0 + m * K, K)], idx_d)
        for j in range(K):
            pltpu.sync_copy(ones, acc.at[idx_d.at[j]], add=True)
        return carry
    lax.fori_loop(0, (EROWS // (NC * NS)) // K, macro, 0)
    plsc.subcore_barrier()
    pltpu.sync_copy(acc.at[pl.ds(s * NP_TILE, NP_TILE)],
                    out.at[pl.ds(c * NPAD + s * NP_TILE, NP_TILE)])


def _pad_tab(t):
    qs = [jnp.pad(t[:, q * 16:(q + 1) * 16], ((0, NPAD - N), (0, 0)))
          for q in range(4)]
    return jnp.concatenate(qs, axis=0)


def kernel(rna, protein, params, edge_index):
    p = params

    src = edge_index[0]
    dst = edge_index[1]
    pad_ids = jnp.full((EPAD - E,), N, i32)
    dst2d = jnp.concatenate([dst, pad_ids]).reshape(EROWS, SUB)
    src2d = jnp.concatenate([src, pad_ids]).reshape(EROWS, SUB)
    zeros16 = jnp.zeros((NP_TILE, 16), f32)

    tc1 = _call_tc(
        _tc1_body, [D, D],
        [(D, H), (8, H), (H, H), (8, H),
         (D, H), (8, H), (H, H), (8, H),
         (H, H), (H, H), (8, H), (H, O), (8, O),
         (O, H), (8, H), (H, H), (O, H), (8, H), (H, H), (8, H)],
        3)
    x0, a0, b0 = tc1(
        rna, protein,
        p['enc_rna_w1'], _tile_bias(p['enc_rna_b1']),
        p['enc_rna_w2'], _tile_bias(p['enc_rna_b2']),
        p['enc_protein_w1'], _tile_bias(p['enc_protein_b1']),
        p['enc_protein_w2'], _tile_bias(p['enc_protein_b2']),
        p['int_w1'][:H], p['int_w1'][H:], _tile_bias(p['int_b1']),
        p['int_w2'], _tile_bias(p['int_b2']),
        p['g0_proj_i_w'][:O], _tile_bias(p['g0_proj_i_b']),
        p['g0_msg_w1'][:H],
        p['g0_proj_j_w'][:O], _tile_bias(p['g0_proj_j_b']),
        p['g0_msg_w1'][H:], _tile_bias(p['g0_msg_b1']))

    cnt = _build_count_kernel()(dst2d, zeros16)
    c0 = cnt[:N]
    c1 = cnt[NPAD:NPAD + N]

    s0 = _build_edge_kernel()(_pad_tab(a0), _pad_tab(b0), dst2d, src2d,
                              zeros16)

    def quarters(sarr):
        return [sarr[q * NPAD:q * NPAD + N] for q in range(4)]

    def w2_quarters(w2):
        return [w2[q * 16:(q + 1) * 16] for q in range(4)]

    tc2 = _call_tc(
        _mid_body_next, [16, 16, 16, 16, 16, 16, 64],
        [(16, H), (16, H), (16, H), (16, H), (8, H),
         (H, H), (O, H), (8, H), (H, O), (8, O),
         (O, H), (8, H), (H, H), (O, H), (8, H), (H, H), (8, H)],
        3)
    x1, a1, b1 = tc2(
        *quarters(s0), c0, c1, x0,
        *w2_quarters(p['g0_msg_w2']), _tile_bias(p['g0_msg_b2']),
        p['g0_upd_w1'][:H], p['g0_upd_w1'][H:], _tile_bias(p['g0_upd_b1']),
        p['g0_upd_w2'], _tile_bias(p['g0_upd_b2']),
        p['g1_proj_i_w'][:O], _tile_bias(p['g1_proj_i_b']),
        p['g1_msg_w1'][:H],
        p['g1_proj_j_w'][:O], _tile_bias(p['g1_proj_j_b']),
        p['g1_msg_w1'][H:], _tile_bias(p['g1_msg_b1']))

    s1 = _build_edge_kernel()(_pad_tab(a1), _pad_tab(b1), dst2d, src2d,
                              zeros16)

    tc3 = _call_tc(
        _mid_body_last, [16, 16, 16, 16, 16, 16, 64],
        [(16, H), (16, H), (16, H), (16, H), (8, H),
         (H, H), (O, H), (8, H), (H, O), (8, O)],
        1)
    x2 = tc3(
        *quarters(s1), c0, c1, x1,
        *w2_quarters(p['g1_msg_w2']), _tile_bias(p['g1_msg_b2']),
        p['g1_upd_w1'][:H], p['g1_upd_w1'][H:], _tile_bias(p['g1_upd_b1']),
        p['g1_upd_w2'], _tile_bias(p['g1_upd_b2']))
    return x2


# speedup vs baseline: 4.8009x; 1.1479x over previous
import functools

import jax
import jax.numpy as jnp
from jax import lax
from jax.experimental import pallas as pl
from jax.experimental.pallas import tpu as pltpu
from jax.experimental.pallas import tpu_sc as plsc

N = 50000
E = 800000
D = 128
H = 64
O = 64

NC = 2
NS = 16
NPAD = 50176
NP_TILE = NPAD // NS
EPAD = 819200
EROWS = EPAD // 128
SUB = 128
K = 8
ROWS_PER_TILE = EROWS // NS
MACROS = ROWS_PER_TILE // K

R = 2000
GRID = N // R

f32 = jnp.float32
i32 = jnp.int32


def _relu(v):
    return jnp.maximum(v, 0.0)


def _b(ref):
    return ref[...][0:1, :]


def _tc1_body(rna, prot, wr1, br1, wr2, br2, wp1, bp1, wp2, bp2,
              iw1a, iw1b, ib1, iw2, ib2,
              piw, pib, mtop, pjw, pjb, mbot, mb1,
              x_out, a_out, b_out):
    er = _relu(rna[...] @ wr1[...] + _b(br1)) @ wr2[...] + _b(br2)
    ep = _relu(prot[...] @ wp1[...] + _b(bp1)) @ wp2[...] + _b(bp2)
    h = _relu(er @ iw1a[...] + ep @ iw1b[...] + _b(ib1))
    x = h @ iw2[...] + _b(ib2)
    x_out[...] = x
    a_out[...] = _relu(x @ piw[...] + _b(pib)) @ mtop[...] + _b(mb1)
    b_out[...] = _relu(x @ pjw[...] + _b(pjb)) @ mbot[...]


def _mid_body_next(s0, s1, s2, s3, c0, c1, x,
                   w20, w21, w22, w23, b2, u1a, u1b, ub1, uw2, ub2,
                   piw, pib, mtop, pjw, pjb, mbot, mb1,
                   x_out, a_out, b_out):
    counts = c0[...][:, 0:1] + c1[...][:, 0:1]
    aggr = (s0[...] @ w20[...] + s1[...] @ w21[...]
            + s2[...] @ w22[...] + s3[...] @ w23[...] + counts * _b(b2))
    uh = _relu(aggr @ u1a[...] + x[...] @ u1b[...] + _b(ub1))
    xn = _relu(uh @ uw2[...] + _b(ub2))
    x_out[...] = xn
    a_out[...] = _relu(xn @ piw[...] + _b(pib)) @ mtop[...] + _b(mb1)
    b_out[...] = _relu(xn @ pjw[...] + _b(pjb)) @ mbot[...]


def _mid_body_last(s0, s1, s2, s3, c0, c1, x,
                   w20, w21, w22, w23, b2, u1a, u1b, ub1, uw2, ub2,
                   x_out):
    counts = c0[...][:, 0:1] + c1[...][:, 0:1]
    aggr = (s0[...] @ w20[...] + s1[...] @ w21[...]
            + s2[...] @ w22[...] + s3[...] @ w23[...] + counts * _b(b2))
    uh = _relu(aggr @ u1a[...] + x[...] @ u1b[...] + _b(ub1))
    x_out[...] = _relu(uh @ uw2[...] + _b(ub2))


def _row_spec(cols):
    return pl.BlockSpec((R, cols), lambda i: (i, 0))


def _full_spec(shape):
    return pl.BlockSpec(shape, lambda i: tuple(0 for _ in shape))


def _tile_bias(b):
    return jnp.tile(b.reshape(1, -1), (8, 1))


def _call_tc(body, row_in_cols, weight_shapes, n_out):
    in_specs = ([_row_spec(c) for c in row_in_cols]
                + [_full_spec(s) for s in weight_shapes])
    out_specs = [_row_spec(64) for _ in range(n_out)]
    out_shape = [jax.ShapeDtypeStruct((N, 64), f32) for _ in range(n_out)]
    return pl.pallas_call(
        body,
        grid=(GRID,),
        in_specs=in_specs,
        out_specs=out_specs if n_out > 1 else out_specs[0],
        out_shape=out_shape if n_out > 1 else out_shape[0],
    )


@functools.lru_cache(maxsize=None)
def _build_edge_kernel():
    mesh = plsc.VectorSubcoreMesh(core_axis_name="c", subcore_axis_name="s",
                                  num_cores=NC, num_subcores=NS)
    return pl.kernel(
        _edge_body,
        out_type=jax.ShapeDtypeStruct((4 * NPAD, 16), f32),
        mesh=mesh,
        compiler_params=pltpu.CompilerParams(use_tc_tiling_on_sc=False),
        scratch_types=[
            [pltpu.VMEM((K, SUB), i32)] * 2,
            [pltpu.VMEM((K, SUB), i32)] * 2,
            [pltpu.VMEM((K, SUB), i32)] * 2,
            [pltpu.VMEM((K * SUB, 16), f32)] * 2,
            [pltpu.VMEM((K * SUB, 16), f32)] * 2,
            pltpu.VMEM_SHARED((NPAD, 16), f32),
            pltpu.SemaphoreType.DMA,
            pltpu.SemaphoreType.DMA,
        ],
    )


def _edge_body(a_tab, b_tab, dst2d, src2d, zeros16, out,
               idx_d, idx_ga, idx_gs, abuf, bbuf, acc, sem_a, sem_b):
    c = lax.axis_index("c")
    s = lax.axis_index("s")
    row0 = s * ROWS_PER_TILE

    for p in range(2):
        qoff = (c * 2 + p) * NPAD

        def fire(rbase, pb, qoff=qoff):
            pltpu.sync_copy(dst2d.at[pl.ds(rbase, K)], idx_d[pb])
            pltpu.sync_copy(src2d.at[pl.ds(rbase, K)], idx_gs[pb])

            @plsc.parallel_loop(0, K)
            def adj(j):
                for k in range(SUB // 16):
                    sl = pl.ds(k * 16, 16)
                    idx_ga[pb][j, sl] = idx_d[pb][j, sl] + qoff
                    idx_gs[pb][j, sl] = idx_gs[pb][j, sl] + qoff

            for j in range(K):
                pltpu.async_copy(a_tab.at[idx_ga[pb].at[j]],
                                 abuf[pb].at[pl.ds(j * SUB, SUB)], sem_a)
                pltpu.async_copy(b_tab.at[idx_gs[pb].at[j]],
                                 bbuf[pb].at[pl.ds(j * SUB, SUB)], sem_b)

        def drain(pb):
            for j in range(K):
                pltpu.make_async_copy(a_tab.at[idx_ga[pb].at[j]],
                                      abuf[pb].at[pl.ds(j * SUB, SUB)],
                                      sem_a).wait()
                pltpu.make_async_copy(b_tab.at[idx_gs[pb].at[j]],
                                      bbuf[pb].at[pl.ds(j * SUB, SUB)],
                                      sem_b).wait()

        def process(pb):
            drain(pb)

            @plsc.parallel_loop(0, K * SUB, unroll=8)
            def comp(r):
                sl = pl.ds(0, 16)
                abuf[pb][r, sl] = jnp.maximum(
                    abuf[pb][r, sl] + bbuf[pb][r, sl], 0.0)

            for j in range(K):
                pltpu.sync_copy(abuf[pb].at[pl.ds(j * SUB, SUB)],
                                acc.at[idx_d[pb].at[j]], add=True)

        pltpu.sync_copy(zeros16, acc.at[pl.ds(s * NP_TILE, NP_TILE)])
        plsc.subcore_barrier()

        fire(row0, 0)

        def pair(mm, carry):
            base = row0 + 2 * mm * K
            fire(base + K, 1)
            process(0)
            nxt = jnp.minimum(base + 2 * K, row0 + (MACROS - 1) * K)
            fire(nxt, 0)
            process(1)
            return carry

        lax.fori_loop(0, MACROS // 2, pair, 0)
        drain(0)

        plsc.subcore_barrier()
        pltpu.sync_copy(acc.at[pl.ds(s * NP_TILE, NP_TILE)],
                        out.at[pl.ds(qoff + s * NP_TILE, NP_TILE)])


@functools.lru_cache(maxsize=None)
def _build_count_kernel():
    mesh = plsc.VectorSubcoreMesh(core_axis_name="c", subcore_axis_name="s",
                                  num_cores=NC, num_subcores=NS)
    return pl.kernel(
        _count_body,
        out_type=jax.ShapeDtypeStruct((2 * NPAD, 16), f32),
        mesh=mesh,
        compiler_params=pltpu.CompilerParams(use_tc_tiling_on_sc=False),
        scratch_types=[
            pltpu.VMEM((K, SUB), i32),
            pltpu.VMEM((SUB, 16), f32),
            pltpu.VMEM_SHARED((NPAD, 16), f32),
        ],
    )


def _count_body(dst2d, zeros16, out, idx_d, ones, acc):
    c = lax.axis_index("c")
    s = lax.axis_index("s")

    pltpu.sync_copy(zeros16, acc.at[pl.ds(s * NP_TILE, NP_TILE)])

    def fill(r, carry):
        ones[r, pl.ds(0, 16)] = jnp.full((16,), 1.0, f32)
        return carry
    lax.fori_loop(0, SUB, fill, 0)
    plsc.subcore_barrier()

    w = c * NS + s
    row0 = w * (EROWS // (NC * NS))

    def macro(m, carry):
        pltpu.sync_copy(dst2d.at[pl.ds(row0 + m * K, K)], idx_d)
        for j in range(K):
            pltpu.sync_copy(ones, acc.at[idx_d.at[j]], add=True)
        return carry
    lax.fori_loop(0, (EROWS // (NC * NS)) // K, macro, 0)
    plsc.subcore_barrier()
    pltpu.sync_copy(acc.at[pl.ds(s * NP_TILE, NP_TILE)],
                    out.at[pl.ds(c * NPAD + s * NP_TILE, NP_TILE)])


def _pad_tab(t):
    qs = [jnp.pad(t[:, q * 16:(q + 1) * 16], ((0, NPAD - N), (0, 0)))
          for q in range(4)]
    return jnp.concatenate(qs, axis=0)


def kernel(rna, protein, params, edge_index):
    p = params

    src = edge_index[0]
    dst = edge_index[1]
    pad_ids = jnp.full((EPAD - E,), N, i32)
    dst2d = jnp.concatenate([dst, pad_ids]).reshape(EROWS, SUB)
    src2d = jnp.concatenate([src, pad_ids]).reshape(EROWS, SUB)
    zeros16 = jnp.zeros((NP_TILE, 16), f32)

    tc1 = _call_tc(
        _tc1_body, [D, D],
        [(D, H), (8, H), (H, H), (8, H),
         (D, H), (8, H), (H, H), (8, H),
         (H, H), (H, H), (8, H), (H, O), (8, O),
         (O, H), (8, H), (H, H), (O, H), (8, H), (H, H), (8, H)],
        3)
    x0, a0, b0 = tc1(
        rna, protein,
        p['enc_rna_w1'], _tile_bias(p['enc_rna_b1']),
        p['enc_rna_w2'], _tile_bias(p['enc_rna_b2']),
        p['enc_protein_w1'], _tile_bias(p['enc_protein_b1']),
        p['enc_protein_w2'], _tile_bias(p['enc_protein_b2']),
        p['int_w1'][:H], p['int_w1'][H:], _tile_bias(p['int_b1']),
        p['int_w2'], _tile_bias(p['int_b2']),
        p['g0_proj_i_w'][:O], _tile_bias(p['g0_proj_i_b']),
        p['g0_msg_w1'][:H],
        p['g0_proj_j_w'][:O], _tile_bias(p['g0_proj_j_b']),
        p['g0_msg_w1'][H:], _tile_bias(p['g0_msg_b1']))

    cnt = _build_count_kernel()(dst2d, zeros16)
    c0 = cnt[:N]
    c1 = cnt[NPAD:NPAD + N]

    s0 = _build_edge_kernel()(_pad_tab(a0), _pad_tab(b0), dst2d, src2d,
                              zeros16)

    def quarters(sarr):
        return [sarr[q * NPAD:q * NPAD + N] for q in range(4)]

    def w2_quarters(w2):
        return [w2[q * 16:(q + 1) * 16] for q in range(4)]

    tc2 = _call_tc(
        _mid_body_next, [16, 16, 16, 16, 16, 16, 64],
        [(16, H), (16, H), (16, H), (16, H), (8, H),
         (H, H), (O, H), (8, H), (H, O), (8, O),
         (O, H), (8, H), (H, H), (O, H), (8, H), (H, H), (8, H)],
        3)
    x1, a1, b1 = tc2(
        *quarters(s0), c0, c1, x0,
        *w2_quarters(p['g0_msg_w2']), _tile_bias(p['g0_msg_b2']),
        p['g0_upd_w1'][:H], p['g0_upd_w1'][H:], _tile_bias(p['g0_upd_b1']),
        p['g0_upd_w2'], _tile_bias(p['g0_upd_b2']),
        p['g1_proj_i_w'][:O], _tile_bias(p['g1_proj_i_b']),
        p['g1_msg_w1'][:H],
        p['g1_proj_j_w'][:O], _tile_bias(p['g1_proj_j_b']),
        p['g1_msg_w1'][H:], _tile_bias(p['g1_msg_b1']))

    s1 = _build_edge_kernel()(_pad_tab(a1), _pad_tab(b1), dst2d, src2d,
                              zeros16)

    tc3 = _call_tc(
        _mid_body_last, [16, 16, 16, 16, 16, 16, 64],
        [(16, H), (16, H), (16, H), (16, H), (8, H),
         (H, H), (O, H), (8, H), (H, O), (8, O)],
        1)
    x2 = tc3(
        *quarters(s1), c0, c1, x1,
        *w2_quarters(p['g1_msg_w2']), _tile_bias(p['g1_msg_b2']),
        p['g1_upd_w1'][:H], p['g1_upd_w1'][H:], _tile_bias(p['g1_upd_b1']),
        p['g1_upd_w2'], _tile_bias(p['g1_upd_b2']))
    return x2


# speedup vs baseline: 7.1664x; 1.4927x over previous
import functools

import jax
import jax.numpy as jnp
from jax import lax
from jax.experimental import pallas as pl
from jax.experimental.pallas import tpu as pltpu
from jax.experimental.pallas import tpu_sc as plsc

N = 50000
E = 800000
D = 128
H = 64
O = 64

NC = 2
NS = 16
NPAD = 50176
NP_TILE = NPAD // NS
EPAD = 819200
EROWS = EPAD // 128
SUB = 128
K = 8
ROWS_PER_TILE = EROWS // NS
MACROS = ROWS_PER_TILE // K

R2C = NP_TILE
GRID2 = NPAD // R2C

f32 = jnp.float32
i32 = jnp.int32


def _relu(v):
    return jnp.maximum(v, 0.0)


def _b(ref):
    return ref[...][0:1, :]


def _tc1_body(rna, prot, wr1, br1, wr2, br2, wp1, bp1, wp2, bp2,
              iw1a, iw1b, ib1, iw2, ib2,
              piw, pib, mtop, pjw, pjb, mbot, mb1,
              x_out, a_out, b_out):
    er = _relu(rna[...] @ wr1[...] + _b(br1)) @ wr2[...] + _b(br2)
    ep = _relu(prot[...] @ wp1[...] + _b(bp1)) @ wp2[...] + _b(bp2)
    h = _relu(er @ iw1a[...] + ep @ iw1b[...] + _b(ib1))
    x = h @ iw2[...] + _b(ib2)
    x_out[...] = x
    a_out[...] = _relu(x @ piw[...] + _b(pib)) @ mtop[...] + _b(mb1)
    b_out[...] = _relu(x @ pjw[...] + _b(pjb)) @ mbot[...]


def _mid_body_next(s0, s1, s2, s3, c0, c1, x,
                   w20, w21, w22, w23, b2, u1a, u1b, ub1, uw2, ub2,
                   piw, pib, mtop, pjw, pjb, mbot, mb1,
                   x_out, a_out, b_out):
    counts = c0[...][:, 0:1] + c1[...][:, 0:1]
    aggr = (s0[...] @ w20[...] + s1[...] @ w21[...]
            + s2[...] @ w22[...] + s3[...] @ w23[...] + counts * _b(b2))
    uh = _relu(aggr @ u1a[...] + x[...] @ u1b[...] + _b(ub1))
    xn = _relu(uh @ uw2[...] + _b(ub2))
    x_out[...] = xn
    a_out[...] = _relu(xn @ piw[...] + _b(pib)) @ mtop[...] + _b(mb1)
    b_out[...] = _relu(xn @ pjw[...] + _b(pjb)) @ mbot[...]


def _mid_body_last(s0, s1, s2, s3, c0, c1, x,
                   w20, w21, w22, w23, b2, u1a, u1b, ub1, uw2, ub2,
                   x_out):
    counts = c0[...][:, 0:1] + c1[...][:, 0:1]
    aggr = (s0[...] @ w20[...] + s1[...] @ w21[...]
            + s2[...] @ w22[...] + s3[...] @ w23[...] + counts * _b(b2))
    uh = _relu(aggr @ u1a[...] + x[...] @ u1b[...] + _b(ub1))
    x_out[...] = _relu(uh @ uw2[...] + _b(ub2))


def _row_spec(cols, off=0):
    return pl.BlockSpec((R2C, cols), lambda i, off=off: (off + i, 0))


def _full_spec(shape):
    return pl.BlockSpec(shape, lambda i: tuple(0 for _ in shape))


def _tile_bias(b):
    return jnp.tile(b.reshape(1, -1), (8, 1))


def _call_tc(body, row_in_specs, weight_shapes, n_out):
    in_specs = list(row_in_specs) + [_full_spec(s) for s in weight_shapes]
    out_specs = [_row_spec(64) for _ in range(n_out)]
    out_shape = [jax.ShapeDtypeStruct((NPAD, 64), f32) for _ in range(n_out)]
    return pl.pallas_call(
        body,
        grid=(GRID2,),
        in_specs=in_specs,
        out_specs=out_specs if n_out > 1 else out_specs[0],
        out_shape=out_shape if n_out > 1 else out_shape[0],
    )


@functools.lru_cache(maxsize=None)
def _build_edge_kernel():
    mesh = plsc.VectorSubcoreMesh(core_axis_name="c", subcore_axis_name="s",
                                  num_cores=NC, num_subcores=NS)
    return pl.kernel(
        _edge_body,
        out_type=jax.ShapeDtypeStruct((4 * NPAD, 16), f32),
        mesh=mesh,
        compiler_params=pltpu.CompilerParams(use_tc_tiling_on_sc=False),
        scratch_types=[
            [pltpu.VMEM((K, SUB), i32)] * 2,
            [pltpu.VMEM((K, SUB), i32)] * 2,
            [pltpu.VMEM((K, SUB), i32)] * 2,
            [pltpu.VMEM((K * SUB, 16), f32)] * 2,
            [pltpu.VMEM((K * SUB, 16), f32)] * 2,
            pltpu.VMEM_SHARED((NPAD, 16), f32),
            pltpu.SemaphoreType.DMA,
            pltpu.SemaphoreType.DMA,
        ],
    )


def _edge_body(a_tab, b_tab, dst2d, src2d, zeros16, out,
               idx_d, idx_ga, idx_gs, abuf, bbuf, acc, sem_a, sem_b):
    c = lax.axis_index("c")
    s = lax.axis_index("s")
    row0 = s * ROWS_PER_TILE

    for p in range(2):
        q = c * 2 + p
        qoff = q * NPAD

        def fire(rbase, pb, q=q):
            pltpu.sync_copy(dst2d.at[pl.ds(rbase, K)], idx_d[pb])
            pltpu.sync_copy(src2d.at[pl.ds(rbase, K)], idx_gs[pb])

            @plsc.parallel_loop(0, K)
            def adj(j):
                for k in range(SUB // 16):
                    sl = pl.ds(k * 16, 16)
                    idx_ga[pb][j, sl] = idx_d[pb][j, sl] * 4 + q
                    idx_gs[pb][j, sl] = idx_gs[pb][j, sl] * 4 + q

            for j in range(K):
                pltpu.async_copy(a_tab.at[idx_ga[pb].at[j]],
                                 abuf[pb].at[pl.ds(j * SUB, SUB)], sem_a)
                pltpu.async_copy(b_tab.at[idx_gs[pb].at[j]],
                                 bbuf[pb].at[pl.ds(j * SUB, SUB)], sem_b)

        def drain(pb):
            for j in range(K):
                pltpu.make_async_copy(a_tab.at[idx_ga[pb].at[j]],
                                      abuf[pb].at[pl.ds(j * SUB, SUB)],
                                      sem_a).wait()
                pltpu.make_async_copy(b_tab.at[idx_gs[pb].at[j]],
                                      bbuf[pb].at[pl.ds(j * SUB, SUB)],
                                      sem_b).wait()

        def process(pb):
            drain(pb)

            @plsc.parallel_loop(0, K * SUB, unroll=8)
            def comp(r):
                sl = pl.ds(0, 16)
                abuf[pb][r, sl] = jnp.maximum(
                    abuf[pb][r, sl] + bbuf[pb][r, sl], 0.0)

            for j in range(K):
                pltpu.sync_copy(abuf[pb].at[pl.ds(j * SUB, SUB)],
                                acc.at[idx_d[pb].at[j]], add=True)

        pltpu.sync_copy(zeros16, acc.at[pl.ds(s * NP_TILE, NP_TILE)])
        plsc.subcore_barrier()

        fire(row0, 0)

        def pair(mm, carry):
            base = row0 + 2 * mm * K
            fire(base + K, 1)
            process(0)
            nxt = jnp.minimum(base + 2 * K, row0 + (MACROS - 1) * K)
            fire(nxt, 0)
            process(1)
            return carry

        lax.fori_loop(0, MACROS // 2, pair, 0)
        drain(0)

        plsc.subcore_barrier()
        pltpu.sync_copy(acc.at[pl.ds(s * NP_TILE, NP_TILE)],
                        out.at[pl.ds(qoff + s * NP_TILE, NP_TILE)])


@functools.lru_cache(maxsize=None)
def _build_count_kernel():
    mesh = plsc.VectorSubcoreMesh(core_axis_name="c", subcore_axis_name="s",
                                  num_cores=NC, num_subcores=NS)
    return pl.kernel(
        _count_body,
        out_type=jax.ShapeDtypeStruct((2 * NPAD, 16), f32),
        mesh=mesh,
        compiler_params=pltpu.CompilerParams(use_tc_tiling_on_sc=False),
        scratch_types=[
            pltpu.VMEM((K, SUB), i32),
            pltpu.VMEM((SUB, 16), f32),
            pltpu.VMEM_SHARED((NPAD, 16), f32),
        ],
    )


def _count_body(dst2d, zeros16, out, idx_d, ones, acc):
    c = lax.axis_index("c")
    s = lax.axis_index("s")

    pltpu.sync_copy(zeros16, acc.at[pl.ds(s * NP_TILE, NP_TILE)])

    def fill(r, carry):
        ones[r, pl.ds(0, 16)] = jnp.full((16,), 1.0, f32)
        return carry
    lax.fori_loop(0, SUB, fill, 0)
    plsc.subcore_barrier()

    w = c * NS + s
    row0 = w * (EROWS // (NC * NS))

    def macro(m, carry):
        pltpu.sync_copy(dst2d.at[pl.ds(row0 + m * K, K)], idx_d)
        for j in range(K):
            pltpu.sync_copy(ones, acc.at[idx_d.at[j]], add=True)
        return carry
    lax.fori_loop(0, (EROWS // (NC * NS)) // K, macro, 0)
    plsc.subcore_barrier()
    pltpu.sync_copy(acc.at[pl.ds(s * NP_TILE, NP_TILE)],
                    out.at[pl.ds(c * NPAD + s * NP_TILE, NP_TILE)])


def kernel(rna, protein, params, edge_index):
    p = params

    src = edge_index[0]
    dst = edge_index[1]
    pad_ids = jnp.full((EPAD - E,), N, i32)
    dst2d = jnp.concatenate([dst, pad_ids]).reshape(EROWS, SUB)
    src2d = jnp.concatenate([src, pad_ids]).reshape(EROWS, SUB)
    zeros16 = jnp.zeros((NP_TILE, 16), f32)

    tc1 = _call_tc(
        _tc1_body, [_row_spec(D), _row_spec(D)],
        [(D, H), (8, H), (H, H), (8, H),
         (D, H), (8, H), (H, H), (8, H),
         (H, H), (H, H), (8, H), (H, O), (8, O),
         (O, H), (8, H), (H, H), (O, H), (8, H), (H, H), (8, H)],
        3)
    x0, a0, b0 = tc1(
        rna, protein,
        p['enc_rna_w1'], _tile_bias(p['enc_rna_b1']),
        p['enc_rna_w2'], _tile_bias(p['enc_rna_b2']),
        p['enc_protein_w1'], _tile_bias(p['enc_protein_b1']),
        p['enc_protein_w2'], _tile_bias(p['enc_protein_b2']),
        p['int_w1'][:H], p['int_w1'][H:], _tile_bias(p['int_b1']),
        p['int_w2'], _tile_bias(p['int_b2']),
        p['g0_proj_i_w'][:O], _tile_bias(p['g0_proj_i_b']),
        p['g0_msg_w1'][:H],
        p['g0_proj_j_w'][:O], _tile_bias(p['g0_proj_j_b']),
        p['g0_msg_w1'][H:], _tile_bias(p['g0_msg_b1']))

    cnt = _build_count_kernel()(dst2d, zeros16)

    def tabs(t):
        return t.reshape(4 * NPAD, 16)

    s0 = _build_edge_kernel()(tabs(a0), tabs(b0), dst2d, src2d, zeros16)

    def w2_quarters(w2):
        return [w2[q * 16:(q + 1) * 16] for q in range(4)]

    mid_row_specs = ([_row_spec(16, off=q * GRID2) for q in range(4)]
                     + [_row_spec(16, off=0), _row_spec(16, off=GRID2),
                        _row_spec(64)])

    tc2 = _call_tc(
        _mid_body_next, mid_row_specs,
        [(16, H), (16, H), (16, H), (16, H), (8, H),
         (H, H), (O, H), (8, H), (H, O), (8, O),
         (O, H), (8, H), (H, H), (O, H), (8, H), (H, H), (8, H)],
        3)
    x1, a1, b1 = tc2(
        s0, s0, s0, s0, cnt, cnt, x0,
        *w2_quarters(p['g0_msg_w2']), _tile_bias(p['g0_msg_b2']),
        p['g0_upd_w1'][:H], p['g0_upd_w1'][H:], _tile_bias(p['g0_upd_b1']),
        p['g0_upd_w2'], _tile_bias(p['g0_upd_b2']),
        p['g1_proj_i_w'][:O], _tile_bias(p['g1_proj_i_b']),
        p['g1_msg_w1'][:H],
        p['g1_proj_j_w'][:O], _tile_bias(p['g1_proj_j_b']),
        p['g1_msg_w1'][H:], _tile_bias(p['g1_msg_b1']))

    s1 = _build_edge_kernel()(tabs(a1), tabs(b1), dst2d, src2d, zeros16)

    tc3 = _call_tc(
        _mid_body_last, mid_row_specs,
        [(16, H), (16, H), (16, H), (16, H), (8, H),
         (H, H), (O, H), (8, H), (H, O), (8, O)],
        1)
    x2 = tc3(
        s1, s1, s1, s1, cnt, cnt, x1,
        *w2_quarters(p['g1_msg_w2']), _tile_bias(p['g1_msg_b2']),
        p['g1_upd_w1'][:H], p['g1_upd_w1'][H:], _tile_bias(p['g1_upd_b1']),
        p['g1_upd_w2'], _tile_bias(p['g1_upd_b2']))
    return x2[:N]


# speedup vs baseline: 7.3450x; 1.0249x over previous
import functools

import jax
import jax.numpy as jnp
from jax import lax
from jax.experimental import pallas as pl
from jax.experimental.pallas import tpu as pltpu
from jax.experimental.pallas import tpu_sc as plsc

N = 50000
E = 800000
D = 128
H = 64
O = 64

NC = 2
NS = 16
NPAD = 50176
NP_TILE = NPAD // NS
EPAD = 819200
EROWS = EPAD // 128
SUB = 128
K = 8
ROWS_PER_TILE = EROWS // NS
MACROS = ROWS_PER_TILE // K

R2C = NP_TILE
GRID2 = NPAD // R2C

f32 = jnp.float32
i32 = jnp.int32


def _relu(v):
    return jnp.maximum(v, 0.0)


def _b(ref):
    return ref[...][0:1, :]


def _tc1_body(rna, prot, wr1, br1, wr2, br2, wp1, bp1, wp2, bp2,
              iw1a, iw1b, ib1, iw2, ib2,
              piw, pib, mtop, pjw, pjb, mbot, mb1,
              x_out, a_out, b_out):
    er = _relu(rna[...] @ wr1[...] + _b(br1)) @ wr2[...] + _b(br2)
    ep = _relu(prot[...] @ wp1[...] + _b(bp1)) @ wp2[...] + _b(bp2)
    h = _relu(er @ iw1a[...] + ep @ iw1b[...] + _b(ib1))
    x = h @ iw2[...] + _b(ib2)
    x_out[...] = x
    a_out[...] = _relu(x @ piw[...] + _b(pib)) @ mtop[...] + _b(mb1)
    b_out[...] = _relu(x @ pjw[...] + _b(pjb)) @ mbot[...]


def _mid_body_next(s0, s1, s2, s3, c0, c1, x,
                   w20, w21, w22, w23, b2, u1a, u1b, ub1, uw2, ub2,
                   piw, pib, mtop, pjw, pjb, mbot, mb1,
                   x_out, a_out, b_out):
    counts = c0[...][:, 0:1] + c1[...][:, 0:1]
    aggr = (s0[...] @ w20[...] + s1[...] @ w21[...]
            + s2[...] @ w22[...] + s3[...] @ w23[...] + counts * _b(b2))
    uh = _relu(aggr @ u1a[...] + x[...] @ u1b[...] + _b(ub1))
    xn = _relu(uh @ uw2[...] + _b(ub2))
    x_out[...] = xn
    a_out[...] = _relu(xn @ piw[...] + _b(pib)) @ mtop[...] + _b(mb1)
    b_out[...] = _relu(xn @ pjw[...] + _b(pjb)) @ mbot[...]


def _mid_body_last(s0, s1, s2, s3, c0, c1, x,
                   w20, w21, w22, w23, b2, u1a, u1b, ub1, uw2, ub2,
                   x_out):
    counts = c0[...][:, 0:1] + c1[...][:, 0:1]
    aggr = (s0[...] @ w20[...] + s1[...] @ w21[...]
            + s2[...] @ w22[...] + s3[...] @ w23[...] + counts * _b(b2))
    uh = _relu(aggr @ u1a[...] + x[...] @ u1b[...] + _b(ub1))
    x_out[...] = _relu(uh @ uw2[...] + _b(ub2))


def _row_spec(cols, off=0):
    return pl.BlockSpec((R2C, cols), lambda i, off=off: (off + i, 0))


def _full_spec(shape):
    return pl.BlockSpec(shape, lambda i: tuple(0 for _ in shape))


def _tile_bias(b):
    return jnp.tile(b.reshape(1, -1), (8, 1))


def _call_tc(body, row_in_specs, weight_shapes, n_out):
    in_specs = list(row_in_specs) + [_full_spec(s) for s in weight_shapes]
    out_specs = [_row_spec(64) for _ in range(n_out)]
    out_shape = [jax.ShapeDtypeStruct((NPAD, 64), f32) for _ in range(n_out)]
    return pl.pallas_call(
        body,
        grid=(GRID2,),
        in_specs=in_specs,
        out_specs=out_specs if n_out > 1 else out_specs[0],
        out_shape=out_shape if n_out > 1 else out_shape[0],
    )


@functools.lru_cache(maxsize=None)
def _build_edge_kernel():
    mesh = plsc.VectorSubcoreMesh(core_axis_name="c", subcore_axis_name="s",
                                  num_cores=NC, num_subcores=NS)
    return pl.kernel(
        _edge_body,
        out_type=jax.ShapeDtypeStruct((4 * NPAD, 16), f32),
        mesh=mesh,
        compiler_params=pltpu.CompilerParams(use_tc_tiling_on_sc=False),
        scratch_types=[
            [pltpu.VMEM((K, SUB), i32)] * 2,
            [pltpu.VMEM((K, SUB), i32)] * 2,
            [pltpu.VMEM((K, SUB), i32)] * 2,
            [pltpu.VMEM((K * SUB, 16), f32)] * 2,
            [pltpu.VMEM((K * SUB, 16), f32)] * 2,
            pltpu.VMEM_SHARED((NPAD, 16), f32),
            pltpu.SemaphoreType.DMA,
            pltpu.SemaphoreType.DMA,
            pltpu.SemaphoreType.DMA,
        ],
    )


def _edge_body(a_tab, b_tab, dst2d, src2d, zeros16, out,
               idx_d, idx_ga, idx_gs, abuf, bbuf, acc, sem_a, sem_b, sem_s):
    c = lax.axis_index("c")
    s = lax.axis_index("s")
    row0 = s * ROWS_PER_TILE

    for p in range(2):
        q = c * 2 + p
        qoff = q * NPAD

        def fire(rbase, pb, q=q):
            pltpu.sync_copy(dst2d.at[pl.ds(rbase, K)], idx_d[pb])
            pltpu.sync_copy(src2d.at[pl.ds(rbase, K)], idx_gs[pb])

            @plsc.parallel_loop(0, K)
            def adj(j):
                for k in range(SUB // 16):
                    sl = pl.ds(k * 16, 16)
                    idx_ga[pb][j, sl] = idx_d[pb][j, sl] * 4 + q
                    idx_gs[pb][j, sl] = idx_gs[pb][j, sl] * 4 + q

            for j in range(K):
                pltpu.async_copy(a_tab.at[idx_ga[pb].at[j]],
                                 abuf[pb].at[pl.ds(j * SUB, SUB)], sem_a)
                pltpu.async_copy(b_tab.at[idx_gs[pb].at[j]],
                                 bbuf[pb].at[pl.ds(j * SUB, SUB)], sem_b)

        def drain(pb):
            for j in range(K):
                pltpu.make_async_copy(a_tab.at[idx_ga[pb].at[j]],
                                      abuf[pb].at[pl.ds(j * SUB, SUB)],
                                      sem_a).wait()
                pltpu.make_async_copy(b_tab.at[idx_gs[pb].at[j]],
                                      bbuf[pb].at[pl.ds(j * SUB, SUB)],
                                      sem_b).wait()

        def process(pb):
            drain(pb)

            @plsc.parallel_loop(0, K * SUB, unroll=8)
            def comp(r):
                sl = pl.ds(0, 16)
                abuf[pb][r, sl] = jnp.maximum(
                    abuf[pb][r, sl] + bbuf[pb][r, sl], 0.0)

            scats = []
            for j in range(K):
                scats.append(pltpu.async_copy(
                    abuf[pb].at[pl.ds(j * SUB, SUB)],
                    acc.at[idx_d[pb].at[j]], sem_s, add=True))
            for sc in scats:
                sc.wait()

        pltpu.sync_copy(zeros16, acc.at[pl.ds(s * NP_TILE, NP_TILE)])
        plsc.subcore_barrier()

        fire(row0, 0)

        def pair(mm, carry):
            base = row0 + 2 * mm * K
            fire(base + K, 1)
            process(0)
            nxt = jnp.minimum(base + 2 * K, row0 + (MACROS - 1) * K)
            fire(nxt, 0)
            process(1)
            return carry

        lax.fori_loop(0, MACROS // 2, pair, 0)
        drain(0)

        plsc.subcore_barrier()
        pltpu.sync_copy(acc.at[pl.ds(s * NP_TILE, NP_TILE)],
                        out.at[pl.ds(qoff + s * NP_TILE, NP_TILE)])


@functools.lru_cache(maxsize=None)
def _build_count_kernel():
    mesh = plsc.VectorSubcoreMesh(core_axis_name="c", subcore_axis_name="s",
                                  num_cores=NC, num_subcores=NS)
    return pl.kernel(
        _count_body,
        out_type=jax.ShapeDtypeStruct((2 * NPAD, 16), f32),
        mesh=mesh,
        compiler_params=pltpu.CompilerParams(use_tc_tiling_on_sc=False),
        scratch_types=[
            pltpu.VMEM((K, SUB), i32),
            pltpu.VMEM((SUB, 16), f32),
            pltpu.VMEM_SHARED((NPAD, 16), f32),
            pltpu.SemaphoreType.DMA,
        ],
    )


def _count_body(dst2d, zeros16, out, idx_d, ones, acc, sem_s):
    c = lax.axis_index("c")
    s = lax.axis_index("s")

    pltpu.sync_copy(zeros16, acc.at[pl.ds(s * NP_TILE, NP_TILE)])

    def fill(r, carry):
        ones[r, pl.ds(0, 16)] = jnp.full((16,), 1.0, f32)
        return carry
    lax.fori_loop(0, SUB, fill, 0)
    plsc.subcore_barrier()

    w = c * NS + s
    row0 = w * (EROWS // (NC * NS))

    def macro(m, carry):
        pltpu.sync_copy(dst2d.at[pl.ds(row0 + m * K, K)], idx_d)
        scats = []
        for j in range(K):
            scats.append(pltpu.async_copy(ones, acc.at[idx_d.at[j]], sem_s,
                                          add=True))
        for sc in scats:
            sc.wait()
        return carry
    lax.fori_loop(0, (EROWS // (NC * NS)) // K, macro, 0)
    plsc.subcore_barrier()
    pltpu.sync_copy(acc.at[pl.ds(s * NP_TILE, NP_TILE)],
                    out.at[pl.ds(c * NPAD + s * NP_TILE, NP_TILE)])


def kernel(rna, protein, params, edge_index):
    p = params

    src = edge_index[0]
    dst = edge_index[1]
    pad_ids = jnp.full((EPAD - E,), N, i32)
    dst2d = jnp.concatenate([dst, pad_ids]).reshape(EROWS, SUB)
    src2d = jnp.concatenate([src, pad_ids]).reshape(EROWS, SUB)
    zeros16 = jnp.zeros((NP_TILE, 16), f32)

    tc1 = _call_tc(
        _tc1_body, [_row_spec(D), _row_spec(D)],
        [(D, H), (8, H), (H, H), (8, H),
         (D, H), (8, H), (H, H), (8, H),
         (H, H), (H, H), (8, H), (H, O), (8, O),
         (O, H), (8, H), (H, H), (O, H), (8, H), (H, H), (8, H)],
        3)
    x0, a0, b0 = tc1(
        rna, protein,
        p['enc_rna_w1'], _tile_bias(p['enc_rna_b1']),
        p['enc_rna_w2'], _tile_bias(p['enc_rna_b2']),
        p['enc_protein_w1'], _tile_bias(p['enc_protein_b1']),
        p['enc_protein_w2'], _tile_bias(p['enc_protein_b2']),
        p['int_w1'][:H], p['int_w1'][H:], _tile_bias(p['int_b1']),
        p['int_w2'], _tile_bias(p['int_b2']),
        p['g0_proj_i_w'][:O], _tile_bias(p['g0_proj_i_b']),
        p['g0_msg_w1'][:H],
        p['g0_proj_j_w'][:O], _tile_bias(p['g0_proj_j_b']),
        p['g0_msg_w1'][H:], _tile_bias(p['g0_msg_b1']))

    cnt = _build_count_kernel()(dst2d, zeros16)

    def tabs(t):
        return t.reshape(4 * NPAD, 16)

    s0 = _build_edge_kernel()(tabs(a0), tabs(b0), dst2d, src2d, zeros16)

    def w2_quarters(w2):
        return [w2[q * 16:(q + 1) * 16] for q in range(4)]

    mid_row_specs = ([_row_spec(16, off=q * GRID2) for q in range(4)]
                     + [_row_spec(16, off=0), _row_spec(16, off=GRID2),
                        _row_spec(64)])

    tc2 = _call_tc(
        _mid_body_next, mid_row_specs,
        [(16, H), (16, H), (16, H), (16, H), (8, H),
         (H, H), (O, H), (8, H), (H, O), (8, O),
         (O, H), (8, H), (H, H), (O, H), (8, H), (H, H), (8, H)],
        3)
    x1, a1, b1 = tc2(
        s0, s0, s0, s0, cnt, cnt, x0,
        *w2_quarters(p['g0_msg_w2']), _tile_bias(p['g0_msg_b2']),
        p['g0_upd_w1'][:H], p['g0_upd_w1'][H:], _tile_bias(p['g0_upd_b1']),
        p['g0_upd_w2'], _tile_bias(p['g0_upd_b2']),
        p['g1_proj_i_w'][:O], _tile_bias(p['g1_proj_i_b']),
        p['g1_msg_w1'][:H],
        p['g1_proj_j_w'][:O], _tile_bias(p['g1_proj_j_b']),
        p['g1_msg_w1'][H:], _tile_bias(p['g1_msg_b1']))

    s1 = _build_edge_kernel()(tabs(a1), tabs(b1), dst2d, src2d, zeros16)

    tc3 = _call_tc(
        _mid_body_last, mid_row_specs,
        [(16, H), (16, H), (16, H), (16, H), (8, H),
         (H, H), (O, H), (8, H), (H, O), (8, O)],
        1)
    x2 = tc3(
        s1, s1, s1, s1, cnt, cnt, x1,
        *w2_quarters(p['g1_msg_w2']), _tile_bias(p['g1_msg_b2']),
        p['g1_upd_w1'][:H], p['g1_upd_w1'][H:], _tile_bias(p['g1_upd_b1']),
        p['g1_upd_w2'], _tile_bias(p['g1_upd_b2']))
    return x2[:N]
